# baseline (speedup 1.0000x reference)
import functools

import jax
import jax.numpy as jnp
from jax import lax
from jax.experimental import pallas as pl
from jax.experimental.pallas import tpu as pltpu
from jax.experimental.pallas import tpu_sc as plsc

NC = 2
NS = 16
IB = 125
RC = 8
NG = 500
F32 = jnp.float32


def _sc_mesh():
    return plsc.VectorSubcoreMesh(core_axis_name="c", subcore_axis_name="s")


_SC_PARAMS = pltpu.CompilerParams(use_tc_tiling_on_sc=False)


def _sc_gather(table, idx2d):
    n, h = table.shape
    h2 = h // NC
    r = idx2d.shape[0]
    rc = 2
    per_t = r // NS
    chunks = per_t // rc
    n2 = chunks // 2
    n_t = n // NS

    @functools.partial(
        pl.kernel,
        out_type=jax.ShapeDtypeStruct((r, IB, h), F32),
        mesh=_sc_mesh(),
        scratch_types=[
            pltpu.VMEM_SHARED((n, h2), F32),
            pltpu.VMEM((2, rc, IB), jnp.int32),
            pltpu.VMEM((2, rc, IB, h2), F32),
            pltpu.SemaphoreType.DMA,
            pltpu.SemaphoreType.DMA,
            pltpu.SemaphoreType.DMA,
        ],
        compiler_params=_SC_PARAMS,
    )
    def k(table_h, idx_h, out_h, table_s, idx_v, rows_v, sem_i0, sem_i1, sem_g):
        c = lax.axis_index("c")
        s = lax.axis_index("s")
        sem_i = (sem_i0, sem_i1)

        pltpu.sync_copy(
            table_h.at[pl.ds(s * n_t, n_t), pl.ds(c * h2, h2)],
            table_s.at[pl.ds(s * n_t, n_t)],
        )
        plsc.subcore_barrier()

        def start(t, b):
            rb = s * per_t + t * rc
            pltpu.async_copy(idx_h.at[pl.ds(rb, rc)], idx_v.at[b], sem_i[b])

        def finish(t, b):
            rb = s * per_t + t * rc
            pltpu.make_async_copy(
                idx_h.at[pl.ds(rb, rc)], idx_v.at[b], sem_i[b]
            ).wait()
            ds_ = [
                pltpu.async_copy(
                    table_s.at[idx_v.at[b].at[j]], rows_v.at[b].at[j], sem_g
                )
                for j in range(rc)
            ]
            for d in ds_:
                d.wait()
            pltpu.sync_copy(
                rows_v.at[b], out_h.at[pl.ds(rb, rc), :, pl.ds(c * h2, h2)]
            )

        start(0, 0)

        def body(p, carry):
            t0 = 2 * p
            start(t0 + 1, 1)
            finish(t0, 0)

            @pl.when(p + 1 < n2)
            def _():
                start(t0 + 2, 0)

            finish(t0 + 1, 1)
            return carry

        lax.fori_loop(0, n2, body, 0)

    return k(table, idx2d)


def _sc_scatter_add(msg3d, idx2d, zeros_h, n):
    r, _, h = msg3d.shape
    h2 = h // NC
    rc = 2
    per_t = r // NS
    chunks = per_t // rc
    n2 = chunks // 2
    n_t = n // NS

    @functools.partial(
        pl.kernel,
        out_type=jax.ShapeDtypeStruct((n, h), F32),
        mesh=_sc_mesh(),
        scratch_types=[
            pltpu.VMEM_SHARED((n, h2), F32),
            pltpu.VMEM((2, rc, IB), jnp.int32),
            pltpu.VMEM((2, rc, IB, h2), F32),
            pltpu.SemaphoreType.DMA,
            pltpu.SemaphoreType.DMA,
            pltpu.SemaphoreType.DMA,
            pltpu.SemaphoreType.DMA,
            pltpu.SemaphoreType.DMA,
        ],
        compiler_params=_SC_PARAMS,
    )
    def k(msg_h, idx_h, z_h, out_h, table_s, idx_v, rows_v,
          sem_i0, sem_i1, sem_m0, sem_m1, sem_s):
        c = lax.axis_index("c")
        s = lax.axis_index("s")
        sem_i = (sem_i0, sem_i1)
        sem_m = (sem_m0, sem_m1)
        pltpu.sync_copy(z_h, table_s.at[pl.ds(s * n_t, n_t)])
        plsc.subcore_barrier()

        def start(t, b):
            rb = s * per_t + t * rc
            pltpu.async_copy(idx_h.at[pl.ds(rb, rc)], idx_v.at[b], sem_i[b])
            pltpu.async_copy(
                msg_h.at[pl.ds(rb, rc), :, pl.ds(c * h2, h2)], rows_v.at[b],
                sem_m[b],
            )

        def finish(t, b):
            rb = s * per_t + t * rc
            pltpu.make_async_copy(
                idx_h.at[pl.ds(rb, rc)], idx_v.at[b], sem_i[b]
            ).wait()
            pltpu.make_async_copy(
                msg_h.at[pl.ds(rb, rc), :, pl.ds(c * h2, h2)], rows_v.at[b],
                sem_m[b],
            ).wait()
            ds_ = [
                pltpu.async_copy(
                    rows_v.at[b].at[j], table_s.at[idx_v.at[b].at[j]], sem_s,
                    add=True,
                )
                for j in range(rc)
            ]
            for d in ds_:
                d.wait()

        start(0, 0)

        def body(p, carry):
            t0 = 2 * p
            start(t0 + 1, 1)
            finish(t0, 0)

            @pl.when(p + 1 < n2)
            def _():
                start(t0 + 2, 0)

            finish(t0 + 1, 1)
            return carry

        lax.fori_loop(0, n2, body, 0)
        plsc.subcore_barrier()
        pltpu.sync_copy(
            table_s.at[pl.ds(s * n_t, n_t)],
            out_h.at[pl.ds(s * n_t, n_t), pl.ds(c * h2, h2)],
        )

    return k(msg3d, idx2d, zeros_h)


def _sc_scatter_gather(msg3d, dst2d, src2d, zeros_h, n):
    r, _, h = msg3d.shape
    h2 = h // NC
    rc = 2
    per_t = r // NS
    chunks = per_t // rc
    n2 = chunks // 2
    n_t = n // NS

    @functools.partial(
        pl.kernel,
        out_type=jax.ShapeDtypeStruct((r, IB, h), F32),
        mesh=_sc_mesh(),
        scratch_types=[
            pltpu.VMEM_SHARED((n, h2), F32),
            pltpu.VMEM((2, rc, IB), jnp.int32),
            pltpu.VMEM((2, rc, IB, h2), F32),
            pltpu.SemaphoreType.DMA,
            pltpu.SemaphoreType.DMA,
            pltpu.SemaphoreType.DMA,
            pltpu.SemaphoreType.DMA,
            pltpu.SemaphoreType.DMA,
        ],
        compiler_params=_SC_PARAMS,
    )
    def k(msg_h, dst_h, src_h, z_h, out_h, table_s, idx_v, rows_v,
          sem_i0, sem_i1, sem_m0, sem_m1, sem_s):
        c = lax.axis_index("c")
        s = lax.axis_index("s")
        sem_i = (sem_i0, sem_i1)
        sem_m = (sem_m0, sem_m1)

        pltpu.sync_copy(z_h, table_s.at[pl.ds(s * n_t, n_t)])
        plsc.subcore_barrier()

        def s_start(t, b):
            rb = s * per_t + t * rc
            pltpu.async_copy(dst_h.at[pl.ds(rb, rc)], idx_v.at[b], sem_i[b])
            pltpu.async_copy(
                msg_h.at[pl.ds(rb, rc), :, pl.ds(c * h2, h2)], rows_v.at[b],
                sem_m[b],
            )

        def s_finish(t, b):
            rb = s * per_t + t * rc
            pltpu.make_async_copy(
                dst_h.at[pl.ds(rb, rc)], idx_v.at[b], sem_i[b]
            ).wait()
            pltpu.make_async_copy(
                msg_h.at[pl.ds(rb, rc), :, pl.ds(c * h2, h2)], rows_v.at[b],
                sem_m[b],
            ).wait()
            ds_ = [
                pltpu.async_copy(
                    rows_v.at[b].at[j], table_s.at[idx_v.at[b].at[j]], sem_s,
                    add=True,
                )
                for j in range(rc)
            ]
            for d in ds_:
                d.wait()

        s_start(0, 0)

        def s_body(p, carry):
            t0 = 2 * p
            s_start(t0 + 1, 1)
            s_finish(t0, 0)

            @pl.when(p + 1 < n2)
            def _():
                s_start(t0 + 2, 0)

            s_finish(t0 + 1, 1)
            return carry

        lax.fori_loop(0, n2, s_body, 0)
        plsc.subcore_barrier()

        def g_start(t, b):
            rb = s * per_t + t * rc
            pltpu.async_copy(src_h.at[pl.ds(rb, rc)], idx_v.at[b], sem_i[b])

        def g_finish(t, b):
            rb = s * per_t + t * rc
            pltpu.make_async_copy(
                src_h.at[pl.ds(rb, rc)], idx_v.at[b], sem_i[b]
            ).wait()
            ds_ = [
                pltpu.async_copy(
                    table_s.at[idx_v.at[b].at[j]], rows_v.at[b].at[j], sem_s
                )
                for j in range(rc)
            ]
            for d in ds_:
                d.wait()
            pltpu.sync_copy(
                rows_v.at[b], out_h.at[pl.ds(rb, rc), :, pl.ds(c * h2, h2)]
            )

        g_start(0, 0)

        def g_body(p, carry):
            t0 = 2 * p
            g_start(t0 + 1, 1)
            g_finish(t0, 0)

            @pl.when(p + 1 < n2)
            def _():
                g_start(t0 + 2, 0)

            g_finish(t0 + 1, 1)
            return carry

        lax.fori_loop(0, n2, g_body, 0)

    return k(msg3d, dst2d, src2d, zeros_h)


def _tc_prep(x, wi_t, wo_t, b_o):
    n, af = x.shape
    h = wi_t.shape[1]
    bn = 2000
    grid = n // bn

    def body(x_ref, wi_ref, wo_ref, bo_ref, xw_ref, xo_ref):
        xb = x_ref[...]
        xw_ref[...] = jnp.dot(xb, wi_ref[...], preferred_element_type=F32)
        xo_ref[...] = (
            jnp.dot(xb, wo_ref[...], preferred_element_type=F32) + bo_ref[...]
        )

    return pl.pallas_call(
        body,
        grid=(grid,),
        in_specs=[
            pl.BlockSpec((bn, af), lambda i: (i, 0)),
            pl.BlockSpec((af, h), lambda i: (0, 0)),
            pl.BlockSpec((af, h), lambda i: (0, 0)),
            pl.BlockSpec((1, h), lambda i: (0, 0)),
        ],
        out_specs=[
            pl.BlockSpec((bn, h), lambda i: (i, 0)),
            pl.BlockSpec((bn, h), lambda i: (i, 0)),
        ],
        out_shape=[
            jax.ShapeDtypeStruct((n, h), F32),
            jax.ShapeDtypeStruct((n, h), F32),
        ],
    )(x, wi_t, wo_t, b_o.reshape(1, h))


def _tc_edge_init(gx2, ea, wib):
    e2, h2 = gx2.shape
    bf = wib.shape[0]
    h = h2 // 2
    bp = 3200
    grid = e2 // bp

    def body(gx_ref, ea_ref, w_ref, mi_ref, m_ref):
        ew = jnp.dot(ea_ref[...], w_ref[...],
                     preferred_element_type=F32)
        ewp = ew.reshape(bp, 2, h).sum(axis=1) * 0.5
        mi = gx_ref[...] + jnp.concatenate([ewp, ewp], axis=1)
        mi_ref[...] = mi
        m_ref[...] = jnp.maximum(mi, 0.0)

    return pl.pallas_call(
        body,
        grid=(grid,),
        in_specs=[
            pl.BlockSpec((bp, h2), lambda i: (i, 0)),
            pl.BlockSpec((2 * bp, bf), lambda i: (i, 0)),
            pl.BlockSpec((bf, h), lambda i: (0, 0)),
        ],
        out_specs=[
            pl.BlockSpec((bp, h2), lambda i: (i, 0)),
            pl.BlockSpec((bp, h2), lambda i: (i, 0)),
        ],
        out_shape=[
            jax.ShapeDtypeStruct((e2, h2), F32),
            jax.ShapeDtypeStruct((e2, h2), F32),
        ],
    )(gx2, ea, wib)


def _tc_edge_update(mi2, gath2, msg2, wh2):
    e2, h2 = mi2.shape
    h = h2 // 2
    bp = 2000
    grid = e2 // bp

    def body(mi_ref, g_ref, *rest):
        if len(rest) == 3:
            m_ref, w_ref, o_ref = rest
            m = m_ref[...]
            mi = mi_ref[...]
        else:
            w_ref, o_ref = rest
            mi = mi_ref[...]
            m = jnp.maximum(mi, 0.0)
        sw = jnp.concatenate([m[:, h:], m[:, :h]], axis=1)
        a = g_ref[...] - sw
        o_ref[...] = jnp.maximum(
            mi + jnp.dot(a, w_ref[...], preferred_element_type=F32), 0.0
        )

    edge_spec = pl.BlockSpec((bp, h2), lambda i: (i, 0))
    w_spec = pl.BlockSpec((h2, h2), lambda i: (0, 0))
    in_specs = [edge_spec, edge_spec]
    args = [mi2, gath2]
    if msg2 is not None:
        in_specs.append(edge_spec)
        args.append(msg2)
    in_specs.append(w_spec)
    args.append(wh2)

    return pl.pallas_call(
        body,
        grid=(grid,),
        in_specs=in_specs,
        out_specs=edge_spec,
        out_shape=jax.ShapeDtypeStruct((e2, h2), F32),
    )(*args)


def _tc_readout(xo, m, gid2d, wo_b):
    n, h = xo.shape
    bn = 2000
    grid = n // bn
    gpad = 512

    def body(xo_ref, m_ref, gid_ref, w_ref, out_ref, acc_ref):
        i = pl.program_id(0)

        @pl.when(i == 0)
        def _():
            acc_ref[...] = jnp.zeros_like(acc_ref)

        hb = jnp.maximum(
            xo_ref[...] + jnp.dot(m_ref[...], w_ref[...], preferred_element_type=F32),
            0.0,
        )
        h2 = jnp.concatenate(
            [hb, jnp.ones((bn, 1), F32), jnp.zeros((bn, 2 * h - 1 - h), F32)], axis=1
        )
        gid = gid_ref[...]
        iota = lax.broadcasted_iota(jnp.int32, (bn, gpad), 1)
        oh = (gid == iota).astype(F32)
        acc_ref[...] += lax.dot_general(
            oh, h2, (((0,), (0,)), ((), ())), preferred_element_type=F32
        )

        @pl.when(i == grid - 1)
        def _():
            acc = acc_ref[...]
            out_ref[...] = acc[:, :h] / jnp.maximum(acc[:, h : h + 1], 1.0)

    return pl.pallas_call(
        body,
        grid=(grid,),
        in_specs=[
            pl.BlockSpec((bn, h), lambda i: (i, 0)),
            pl.BlockSpec((bn, h), lambda i: (i, 0)),
            pl.BlockSpec((bn, 1), lambda i: (i, 0)),
            pl.BlockSpec((h, h), lambda i: (0, 0)),
        ],
        out_specs=pl.BlockSpec((gpad, h), lambda i: (0, 0)),
        out_shape=jax.ShapeDtypeStruct((gpad, h), F32),
        scratch_shapes=[pltpu.VMEM((gpad, 2 * h), F32)],
    )(xo, m, gid2d, wo_b)


def kernel(x, edge_index, edge_attr, graph_ids, W_i, W_h, W_o, b_o):
    n, af = x.shape
    e, bf = edge_attr.shape
    h = W_h.shape[0]
    depth = 3

    src2d = edge_index[0].reshape(e // IB, IB)
    dst2d = edge_index[1].reshape(e // IB, IB)
    zeros_h = jnp.zeros((n // NS, h // NC), F32)

    wh2 = (
        jnp.zeros((2 * h, 2 * h), F32)
        .at[:h, :h].set(W_h)
        .at[h:, h:].set(W_h)
    )

    wib = W_i[af:]

    xw, xo = _tc_prep(x, W_i[:af], W_o[:af], b_o)
    gx2 = _sc_gather(xw, src2d).reshape(e // 2, 2 * h)
    mi2, msg2 = _tc_edge_init(gx2, edge_attr, wib)

    for it in range(depth - 1):
        gath2 = _sc_scatter_gather(
            msg2.reshape(e // IB, IB, h), dst2d, src2d, zeros_h, n
        ).reshape(e // 2, 2 * h)
        msg2 = _tc_edge_update(mi2, gath2, msg2 if it > 0 else None, wh2)

    m = _sc_scatter_add(msg2.reshape(e // IB, IB, h), dst2d, zeros_h, n)
    g = _tc_readout(xo, m, graph_ids.reshape(n, 1), W_o[af:])
    return g[:NG]


# speedup vs baseline: 1.0735x; 1.0735x over previous
import functools

import jax
import jax.numpy as jnp
from jax import lax
from jax.experimental import pallas as pl
from jax.experimental.pallas import tpu as pltpu
from jax.experimental.pallas import tpu_sc as plsc

NC = 2
NS = 16
IB = 125
RC = 8
NG = 500
F32 = jnp.float32


def _sc_mesh():
    return plsc.VectorSubcoreMesh(core_axis_name="c", subcore_axis_name="s")


_SC_PARAMS = pltpu.CompilerParams(use_tc_tiling_on_sc=False)


def _sc_gather(table, idx2d):
    n, h = table.shape
    h2 = h // NC
    r = idx2d.shape[0]
    rc = 2
    per_t = r // NS
    chunks = per_t // rc
    n2 = chunks // 2
    n_t = n // NS

    @functools.partial(
        pl.kernel,
        out_type=jax.ShapeDtypeStruct((r, IB, h), F32),
        mesh=_sc_mesh(),
        scratch_types=[
            pltpu.VMEM_SHARED((n, h2), F32),
            pltpu.VMEM((2, rc, IB), jnp.int32),
            pltpu.VMEM((2, rc, IB, h2), F32),
            pltpu.SemaphoreType.DMA,
            pltpu.SemaphoreType.DMA,
            pltpu.SemaphoreType.DMA,
        ],
        compiler_params=_SC_PARAMS,
    )
    def k(table_h, idx_h, out_h, table_s, idx_v, rows_v, sem_i0, sem_i1, sem_g):
        c = lax.axis_index("c")
        s = lax.axis_index("s")
        sem_i = (sem_i0, sem_i1)

        pltpu.sync_copy(
            table_h.at[pl.ds(s * n_t, n_t), pl.ds(c * h2, h2)],
            table_s.at[pl.ds(s * n_t, n_t)],
        )
        plsc.subcore_barrier()

        def start(t, b):
            rb = s * per_t + t * rc
            pltpu.async_copy(idx_h.at[pl.ds(rb, rc)], idx_v.at[b], sem_i[b])

        def finish(t, b):
            rb = s * per_t + t * rc
            pltpu.make_async_copy(
                idx_h.at[pl.ds(rb, rc)], idx_v.at[b], sem_i[b]
            ).wait()
            ds_ = [
                pltpu.async_copy(
                    table_s.at[idx_v.at[b].at[j]], rows_v.at[b].at[j], sem_g
                )
                for j in range(rc)
            ]
            for d in ds_:
                d.wait()
            pltpu.sync_copy(
                rows_v.at[b], out_h.at[pl.ds(rb, rc), :, pl.ds(c * h2, h2)]
            )

        start(0, 0)

        def body(p, carry):
            t0 = 2 * p
            start(t0 + 1, 1)
            finish(t0, 0)

            @pl.when(p + 1 < n2)
            def _():
                start(t0 + 2, 0)

            finish(t0 + 1, 1)
            return carry

        lax.fori_loop(0, n2, body, 0)

    return k(table, idx2d)


def _sc_scatter_add(msg3d, idx2d, zeros_h, n):
    r, _, h = msg3d.shape
    h2 = h // NC
    rc = 2
    per_t = r // NS
    chunks = per_t // rc
    n2 = chunks // 2
    n_t = n // NS

    @functools.partial(
        pl.kernel,
        out_type=jax.ShapeDtypeStruct((n, h), F32),
        mesh=_sc_mesh(),
        scratch_types=[
            pltpu.VMEM_SHARED((n, h2), F32),
            pltpu.VMEM((2, rc, IB), jnp.int32),
            pltpu.VMEM((2, rc, IB, h2), F32),
            pltpu.SemaphoreType.DMA,
            pltpu.SemaphoreType.DMA,
            pltpu.SemaphoreType.DMA,
            pltpu.SemaphoreType.DMA,
            pltpu.SemaphoreType.DMA,
        ],
        compiler_params=_SC_PARAMS,
    )
    def k(msg_h, idx_h, z_h, out_h, table_s, idx_v, rows_v,
          sem_i0, sem_i1, sem_m0, sem_m1, sem_s):
        c = lax.axis_index("c")
        s = lax.axis_index("s")
        sem_i = (sem_i0, sem_i1)
        sem_m = (sem_m0, sem_m1)
        pltpu.sync_copy(z_h, table_s.at[pl.ds(s * n_t, n_t)])
        plsc.subcore_barrier()

        def start(t, b):
            rb = s * per_t + t * rc
            pltpu.async_copy(idx_h.at[pl.ds(rb, rc)], idx_v.at[b], sem_i[b])
            pltpu.async_copy(
                msg_h.at[pl.ds(rb, rc), :, pl.ds(c * h2, h2)], rows_v.at[b],
                sem_m[b],
            )

        def finish(t, b):
            rb = s * per_t + t * rc
            pltpu.make_async_copy(
                idx_h.at[pl.ds(rb, rc)], idx_v.at[b], sem_i[b]
            ).wait()
            pltpu.make_async_copy(
                msg_h.at[pl.ds(rb, rc), :, pl.ds(c * h2, h2)], rows_v.at[b],
                sem_m[b],
            ).wait()
            ds_ = [
                pltpu.async_copy(
                    rows_v.at[b].at[j], table_s.at[idx_v.at[b].at[j]], sem_s,
                    add=True,
                )
                for j in range(rc)
            ]
            for d in ds_:
                d.wait()

        start(0, 0)

        def body(p, carry):
            t0 = 2 * p
            start(t0 + 1, 1)
            finish(t0, 0)

            @pl.when(p + 1 < n2)
            def _():
                start(t0 + 2, 0)

            finish(t0 + 1, 1)
            return carry

        lax.fori_loop(0, n2, body, 0)
        plsc.subcore_barrier()
        pltpu.sync_copy(
            table_s.at[pl.ds(s * n_t, n_t)],
            out_h.at[pl.ds(s * n_t, n_t), pl.ds(c * h2, h2)],
        )

    return k(msg3d, idx2d, zeros_h)


def _sc_scatter_gather(msg3d, dst2d, src2d, zeros_h, n):
    r, _, h = msg3d.shape
    h2 = h // NC
    rc = 2
    per_t = r // NS
    chunks = per_t // rc
    n2 = chunks // 2
    n_t = n // NS

    @functools.partial(
        pl.kernel,
        out_type=jax.ShapeDtypeStruct((r, IB, h), F32),
        mesh=_sc_mesh(),
        scratch_types=[
            pltpu.VMEM_SHARED((n, h2), F32),
            pltpu.VMEM((2, rc, IB), jnp.int32),
            pltpu.VMEM((2, rc, IB, h2), F32),
            pltpu.SemaphoreType.DMA,
            pltpu.SemaphoreType.DMA,
            pltpu.SemaphoreType.DMA,
            pltpu.SemaphoreType.DMA,
            pltpu.SemaphoreType.DMA,
        ],
        compiler_params=_SC_PARAMS,
    )
    def k(msg_h, dst_h, src_h, z_h, out_h, table_s, idx_v, rows_v,
          sem_i0, sem_i1, sem_m0, sem_m1, sem_s):
        c = lax.axis_index("c")
        s = lax.axis_index("s")
        sem_i = (sem_i0, sem_i1)
        sem_m = (sem_m0, sem_m1)

        pltpu.sync_copy(z_h, table_s.at[pl.ds(s * n_t, n_t)])
        plsc.subcore_barrier()

        def s_start(t, b):
            rb = s * per_t + t * rc
            pltpu.async_copy(dst_h.at[pl.ds(rb, rc)], idx_v.at[b], sem_i[b])
            pltpu.async_copy(
                msg_h.at[pl.ds(rb, rc), :, pl.ds(c * h2, h2)], rows_v.at[b],
                sem_m[b],
            )

        def s_finish(t, b):
            rb = s * per_t + t * rc
            pltpu.make_async_copy(
                dst_h.at[pl.ds(rb, rc)], idx_v.at[b], sem_i[b]
            ).wait()
            pltpu.make_async_copy(
                msg_h.at[pl.ds(rb, rc), :, pl.ds(c * h2, h2)], rows_v.at[b],
                sem_m[b],
            ).wait()
            ds_ = [
                pltpu.async_copy(
                    rows_v.at[b].at[j], table_s.at[idx_v.at[b].at[j]], sem_s,
                    add=True,
                )
                for j in range(rc)
            ]
            for d in ds_:
                d.wait()

        s_start(0, 0)

        def s_body(p, carry):
            t0 = 2 * p
            s_start(t0 + 1, 1)
            s_finish(t0, 0)

            @pl.when(p + 1 < n2)
            def _():
                s_start(t0 + 2, 0)

            s_finish(t0 + 1, 1)
            return carry

        lax.fori_loop(0, n2, s_body, 0)
        plsc.subcore_barrier()

        def g_start(t, b):
            rb = s * per_t + t * rc
            pltpu.async_copy(src_h.at[pl.ds(rb, rc)], idx_v.at[b], sem_i[b])

        def g_finish(t, b):
            rb = s * per_t + t * rc
            pltpu.make_async_copy(
                src_h.at[pl.ds(rb, rc)], idx_v.at[b], sem_i[b]
            ).wait()
            ds_ = [
                pltpu.async_copy(
                    table_s.at[idx_v.at[b].at[j]], rows_v.at[b].at[j], sem_s
                )
                for j in range(rc)
            ]
            for d in ds_:
                d.wait()
            pltpu.sync_copy(
                rows_v.at[b], out_h.at[pl.ds(rb, rc), :, pl.ds(c * h2, h2)]
            )

        g_start(0, 0)

        def g_body(p, carry):
            t0 = 2 * p
            g_start(t0 + 1, 1)
            g_finish(t0, 0)

            @pl.when(p + 1 < n2)
            def _():
                g_start(t0 + 2, 0)

            g_finish(t0 + 1, 1)
            return carry

        lax.fori_loop(0, n2, g_body, 0)

    return k(msg3d, dst2d, src2d, zeros_h)


def _tc_prep(x, wi_t, wo_t, b_o):
    n, af = x.shape
    h = wi_t.shape[1]
    bn = 2000
    grid = n // bn

    def body(x_ref, wi_ref, wo_ref, bo_ref, xw_ref, xo_ref):
        xb = x_ref[...]
        xw_ref[...] = jnp.dot(xb, wi_ref[...], preferred_element_type=F32)
        xo_ref[...] = (
            jnp.dot(xb, wo_ref[...], preferred_element_type=F32) + bo_ref[...]
        )

    return pl.pallas_call(
        body,
        grid=(grid,),
        in_specs=[
            pl.BlockSpec((bn, af), lambda i: (i, 0)),
            pl.BlockSpec((af, h), lambda i: (0, 0)),
            pl.BlockSpec((af, h), lambda i: (0, 0)),
            pl.BlockSpec((1, h), lambda i: (0, 0)),
        ],
        out_specs=[
            pl.BlockSpec((bn, h), lambda i: (i, 0)),
            pl.BlockSpec((bn, h), lambda i: (i, 0)),
        ],
        out_shape=[
            jax.ShapeDtypeStruct((n, h), F32),
            jax.ShapeDtypeStruct((n, h), F32),
        ],
    )(x, wi_t, wo_t, b_o.reshape(1, h))


def _tc_edge_init(gx2, ea2, wib2):
    e2, h2 = gx2.shape
    bf2 = ea2.shape[1]
    bp = 2000
    grid = e2 // bp

    def body(gx_ref, ea_ref, w_ref, mi_ref, m_ref):
        mi = gx_ref[...] + jnp.dot(ea_ref[...], w_ref[...],
                                   preferred_element_type=F32)
        mi_ref[...] = mi
        m_ref[...] = jnp.maximum(mi, 0.0)

    return pl.pallas_call(
        body,
        grid=(grid,),
        in_specs=[
            pl.BlockSpec((bp, h2), lambda i: (i, 0)),
            pl.BlockSpec((bp, bf2), lambda i: (i, 0)),
            pl.BlockSpec((bf2, h2), lambda i: (0, 0)),
        ],
        out_specs=[
            pl.BlockSpec((bp, h2), lambda i: (i, 0)),
            pl.BlockSpec((bp, h2), lambda i: (i, 0)),
        ],
        out_shape=[
            jax.ShapeDtypeStruct((e2, h2), F32),
            jax.ShapeDtypeStruct((e2, h2), F32),
        ],
    )(gx2, ea2, wib2)


def _tc_edge_update(mi2, gath2, msg2, wh2):
    e2, h2 = mi2.shape
    h = h2 // 2
    bp = 2000
    grid = e2 // bp

    def body(mi_ref, g_ref, *rest):
        if len(rest) == 3:
            m_ref, w_ref, o_ref = rest
            m = m_ref[...]
            mi = mi_ref[...]
        else:
            w_ref, o_ref = rest
            mi = mi_ref[...]
            m = jnp.maximum(mi, 0.0)
        sw = jnp.concatenate([m[:, h:], m[:, :h]], axis=1)
        a = g_ref[...] - sw
        o_ref[...] = jnp.maximum(
            mi + jnp.dot(a, w_ref[...], preferred_element_type=F32), 0.0
        )

    edge_spec = pl.BlockSpec((bp, h2), lambda i: (i, 0))
    w_spec = pl.BlockSpec((h2, h2), lambda i: (0, 0))
    in_specs = [edge_spec, edge_spec]
    args = [mi2, gath2]
    if msg2 is not None:
        in_specs.append(edge_spec)
        args.append(msg2)
    in_specs.append(w_spec)
    args.append(wh2)

    return pl.pallas_call(
        body,
        grid=(grid,),
        in_specs=in_specs,
        out_specs=edge_spec,
        out_shape=jax.ShapeDtypeStruct((e2, h2), F32),
    )(*args)


def _tc_readout(xo, m, gid2d, wo_b):
    n, h = xo.shape
    bn = 2000
    grid = n // bn
    gpad = 512

    def body(xo_ref, m_ref, gid_ref, w_ref, out_ref, acc_ref):
        i = pl.program_id(0)

        @pl.when(i == 0)
        def _():
            acc_ref[...] = jnp.zeros_like(acc_ref)

        hb = jnp.maximum(
            xo_ref[...] + jnp.dot(m_ref[...], w_ref[...], preferred_element_type=F32),
            0.0,
        )
        h2 = jnp.concatenate(
            [hb, jnp.ones((bn, 1), F32), jnp.zeros((bn, 2 * h - 1 - h), F32)], axis=1
        )
        gid = gid_ref[...]
        iota = lax.broadcasted_iota(jnp.int32, (bn, gpad), 1)
        oh = (gid == iota).astype(F32)
        acc_ref[...] += lax.dot_general(
            oh, h2, (((0,), (0,)), ((), ())), preferred_element_type=F32
        )

        @pl.when(i == grid - 1)
        def _():
            acc = acc_ref[...]
            out_ref[...] = acc[:, :h] / jnp.maximum(acc[:, h : h + 1], 1.0)

    return pl.pallas_call(
        body,
        grid=(grid,),
        in_specs=[
            pl.BlockSpec((bn, h), lambda i: (i, 0)),
            pl.BlockSpec((bn, h), lambda i: (i, 0)),
            pl.BlockSpec((bn, 1), lambda i: (i, 0)),
            pl.BlockSpec((h, h), lambda i: (0, 0)),
        ],
        out_specs=pl.BlockSpec((gpad, h), lambda i: (0, 0)),
        out_shape=jax.ShapeDtypeStruct((gpad, h), F32),
        scratch_shapes=[pltpu.VMEM((gpad, 2 * h), F32)],
    )(xo, m, gid2d, wo_b)


def kernel(x, edge_index, edge_attr, graph_ids, W_i, W_h, W_o, b_o):
    n, af = x.shape
    e, bf = edge_attr.shape
    h = W_h.shape[0]
    depth = 3

    src2d = edge_index[0].reshape(e // IB, IB)
    dst2d = edge_index[1].reshape(e // IB, IB)
    zeros_h = jnp.zeros((n // NS, h // NC), F32)

    wh2 = (
        jnp.zeros((2 * h, 2 * h), F32)
        .at[:h, :h].set(W_h)
        .at[h:, h:].set(W_h)
    )

    wib2 = (
        jnp.zeros((2 * bf, 2 * h), F32)
        .at[:bf, :h].set(W_i[af:])
        .at[bf:, h:].set(W_i[af:])
    )

    xw, xo = _tc_prep(x, W_i[:af], W_o[:af], b_o)
    gx2 = _sc_gather(xw, src2d).reshape(e // 2, 2 * h)
    mi2, msg2 = _tc_edge_init(gx2, edge_attr.reshape(e // 2, 2 * bf), wib2)

    for it in range(depth - 1):
        gath2 = _sc_scatter_gather(
            msg2.reshape(e // IB, IB, h), dst2d, src2d, zeros_h, n
        ).reshape(e // 2, 2 * h)
        msg2 = _tc_edge_update(mi2, gath2, msg2 if it > 0 else None, wh2)

    m = _sc_scatter_add(msg2.reshape(e // IB, IB, h), dst2d, zeros_h, n)
    g = _tc_readout(xo, m, graph_ids.reshape(n, 1), W_o[af:])
    return g[:NG]
